# baseline (speedup 1.0000x reference)
import functools

import jax
import jax.numpy as jnp
from jax import lax
from jax.experimental import pallas as pl
from jax.experimental.pallas import tpu as pltpu
from jax.experimental.pallas import tpu_sc as plsc

D = 128
S = 50
B = 1024
N = B * S

_NC = 2
_NS = 16
_NW = _NC * _NS
_BPW = N // _NW
_CH = 400
_NCHUNK = _BPW // _CH


def _sc_gather_all(pro_embed, skill_embed, change, diff_mat, np_idx, np_hi,
                   ns_idx):
    mesh = plsc.VectorSubcoreMesh(core_axis_name="c", subcore_axis_name="s")

    @functools.partial(
        pl.kernel,
        mesh=mesh,
        out_type=(
            jax.ShapeDtypeStruct((N, D), jnp.float32),
            jax.ShapeDtypeStruct((N, D), jnp.float32),
            jax.ShapeDtypeStruct((N, D), jnp.float32),
            jax.ShapeDtypeStruct((N, D), jnp.float32),
        ),
        scratch_types=[
            pltpu.VMEM((_BPW,), jnp.int32),
            pltpu.VMEM((_BPW,), jnp.int32),
            pltpu.VMEM((_CH, D), jnp.float32),
            pltpu.VMEM((_CH, D), jnp.float32),
            pltpu.SemaphoreType.DMA,
            pltpu.SemaphoreType.DMA,
        ],
    )
    def k(pro_hbm, skill_hbm, change_hbm, diff_hbm, npi_hbm, nphi_hbm, nsi_hbm,
          pro_out, skill_out, change_out, diff_out, idxp_v, idxs_v,
          rows0_v, rows1_v, sem0, sem1):
        wid = lax.axis_index("s") * _NC + lax.axis_index("c")
        base = wid * _BPW

        pltpu.sync_copy(npi_hbm.at[pl.ds(base, _BPW)], idxp_v)
        pltpu.sync_copy(nsi_hbm.at[pl.ds(base, _BPW)], idxs_v)

        work = []
        for ci in range(_NCHUNK):
            work.append((pro_hbm, idxp_v, pro_out, ci))
            work.append((skill_hbm, idxs_v, skill_out, ci))
            work.append((change_hbm, idxs_v, change_out, ci))
        bufs = (rows0_v, rows1_v)
        sems = (sem0, sem1)

        def start(i):
            tbl, idx, _, ci = work[i]
            pltpu.async_copy(tbl.at[idx.at[pl.ds(ci * _CH, _CH)]],
                             bufs[i % 2], sems[i % 2])

        start(0)
        for i in range(len(work)):
            if i + 1 < len(work):
                start(i + 1)
            tbl, idx, out, ci = work[i]
            pltpu.make_async_copy(tbl.at[idx.at[pl.ds(ci * _CH, _CH)]],
                                  bufs[i % 2], sems[i % 2]).wait()
            pltpu.sync_copy(bufs[i % 2], out.at[pl.ds(base + ci * _CH, _CH)])

        pltpu.sync_copy(nphi_hbm.at[pl.ds(base, _BPW)], idxp_v)
        for ci in range(_NCHUNK):
            off = ci * _CH
            idx_c = idxp_v.at[pl.ds(off, _CH)]
            pltpu.async_copy(diff_hbm.at[idx_c], bufs[ci % 2], sems[ci % 2])
            if ci > 0:
                poff = (ci - 1) * _CH
                pltpu.make_async_copy(
                    diff_hbm.at[idxp_v.at[pl.ds(poff, _CH)]],
                    bufs[(ci - 1) % 2], sems[(ci - 1) % 2]).wait()
                pltpu.sync_copy(bufs[(ci - 1) % 2],
                                diff_out.at[pl.ds(base + poff, _CH)])
        last = _NCHUNK - 1
        pltpu.make_async_copy(diff_hbm.at[idxp_v.at[pl.ds(last * _CH, _CH)]],
                              bufs[last % 2], sems[last % 2]).wait()
        pltpu.sync_copy(bufs[last % 2],
                        diff_out.at[pl.ds(base + last * _CH, _CH)])

    return k(pro_embed, skill_embed, change, diff_mat, np_idx, np_hi, ns_idx)


_BB = 128
_NB = B // _BB
_TPAD = 56


def _scan_kernel(pro_ref, skill_ref, change_ref, diffrow_ref, lo_ref, na_ref,
                 np_ref, ns_ref, aet_ref, tet_ref, lst_ref, ps0t_ref, ss0t_ref,
                 wpfa_ref, wpfb_ref, bpf_ref, wsfa_ref, wsfb_ref, bsf_ref,
                 wafa_ref, wafb_ref, baf_ref, wpsa_ref, wpsb_ref, bps_ref,
                 wssa_ref, wssb_ref, bss_ref, wasa_ref, wasb_ref, bas_ref,
                 w1a_ref, w1b_ref, w1c_ref, w1d_ref, b1_ref, w2_ref, b2_ref,
                 out_ref, histp_ref, hists_ref, lbp_scr, lbs_scr, pacc_scr):
    f32 = jnp.float32
    i32 = jnp.int32
    dot = functools.partial(jnp.dot, preferred_element_type=f32)

    np_all = np_ref[0]
    ns_all = ns_ref[0]

    jjj = lax.broadcasted_iota(i32, (S, S, _BB), 0)
    ttt = lax.broadcasted_iota(i32, (S, S, _BB), 1)
    eqp = (np_all[:, None, :] == np_all[None, :, :]) & (jjj < ttt)
    eqs = (ns_all[:, None, :] == ns_all[None, :, :]) & (jjj < ttt)
    lbp_scr[...] = jnp.max(jnp.where(eqp, jjj, 0), axis=0)
    lbs_scr[...] = jnp.max(jnp.where(eqs, jjj, 0), axis=0)

    tp_tab = dot(wpfb_ref[...], tet_ref[:, 0:_TPAD])
    ts_tab = dot(wsfb_ref[...], tet_ref[:, 0:_TPAD])
    caf = dot(wafb_ref[...], tet_ref[:, 1:2]) + baf_ref[...]

    a0 = aet_ref[:, 0:1]
    a1 = aet_ref[:, 1:2]

    jj_s1b = lax.broadcasted_iota(i32, (S, 1, _BB), 0)
    sub56 = lax.broadcasted_iota(i32, (_TPAD, _BB), 0)
    subd = lax.broadcasted_iota(i32, (D, _BB), 0)

    histp_ref[...] = jnp.zeros((S, D, _BB), f32)
    hists_ref[...] = jnp.zeros((S, D, _BB), f32)
    histp_ref[0] = jnp.broadcast_to(ps0t_ref[...], (D, _BB))
    hists_ref[0] = jnp.broadcast_to(ss0t_ref[...], (D, _BB))
    alls0 = jnp.broadcast_to(lst_ref[...], (D, _BB))

    def substep(t, alls, lbps, lbss, lbpt_row, lbst_row):
        ohp = (sub56 == (t - lbpt_row)).astype(f32)
        ohs = (sub56 == (t - lbst_row)).astype(f32)

        lbps = lbps * jax.nn.sigmoid(
            dot(wpfa_ref[...], lbps) + dot(tp_tab, ohp) + bpf_ref[...])
        lbss = lbss * jax.nn.sigmoid(
            dot(wsfa_ref[...], lbss) + dot(ts_tab, ohs) + bsf_ref[...])
        lbas = alls * jax.nn.sigmoid(dot(wafa_ref[...], alls) + caf)

        pro_t = jnp.transpose(pro_ref[pl.ds(t, 1)][0])
        skill_t = jnp.transpose(skill_ref[pl.ds(t, 1)][0])
        change_t = jnp.transpose(change_ref[pl.ds(t, 1)][0])
        drow_t = jnp.transpose(diffrow_ref[pl.ds(t, 1)][0])
        lo_row = lo_ref[pl.ds(0, 1), pl.ds(t, 1), :][0]
        diff_row = jnp.sum(jnp.where(subd == lo_row, drow_t, 0.0),
                           axis=0, keepdims=True)
        na_row = na_ref[pl.ds(0, 1), pl.ds(t, 1), :][0]
        npe = pro_t + skill_t + diff_row * change_t
        nx = npe + a0 + na_row * (a1 - a0)

        h = jax.nn.relu(dot(w1a_ref[...], lbas) + dot(w1b_ref[...], lbps)
                        + dot(w1c_ref[...], lbss) + dot(w1d_ref[...], npe)
                        + b1_ref[...])
        logit = jnp.sum(h * w2_ref[...], axis=0, keepdims=True) + b2_ref[...]
        pacc_scr[pl.ds(t, 1)] = jax.nn.sigmoid(logit)

        alls_new = lbas + jnp.tanh(
            dot(wasa_ref[...], lbas) + dot(wasb_ref[...], nx) + bas_ref[...])
        ips = lbps + jnp.tanh(
            dot(wpsa_ref[...], lbps) + dot(wpsb_ref[...], nx) + bps_ref[...])
        iss = lbss + jnp.tanh(
            dot(wssa_ref[...], lbss) + dot(wssb_ref[...], nx) + bss_ref[...])
        return alls_new, ips, iss

    def body2(i, alls, jmax):
        t0 = 2 * i
        t1 = t0 + 1
        lbp0 = lbp_scr[pl.ds(t0, 1)]
        lbs0 = lbs_scr[pl.ds(t0, 1)]
        lbp1 = lbp_scr[pl.ds(t1, 1)]
        lbs1 = lbs_scr[pl.ds(t1, 1)]
        m0p = (jj_s1b[0:jmax] == lbp0).astype(f32)
        m0s = (jj_s1b[0:jmax] == lbs0).astype(f32)
        m1p = (jj_s1b[0:jmax] == lbp1).astype(f32)
        m1s = (jj_s1b[0:jmax] == lbs1).astype(f32)
        lbps0 = jnp.sum(histp_ref[0:jmax] * m0p, axis=0)
        lbss0 = jnp.sum(hists_ref[0:jmax] * m0s, axis=0)
        raw1p = jnp.sum(histp_ref[0:jmax] * m1p, axis=0)
        raw1s = jnp.sum(hists_ref[0:jmax] * m1s, axis=0)

        alls, ips0, iss0 = substep(t0, alls, lbps0, lbss0, lbp0, lbs0)

        hit_p = (lbp1 == t0).astype(f32)
        hit_s = (lbs1 == t0).astype(f32)
        lbps1 = raw1p * (1.0 - hit_p) + ips0 * hit_p
        lbss1 = raw1s * (1.0 - hit_s) + iss0 * hit_s
        alls, ips1, iss1 = substep(t1, alls, lbps1, lbss1, lbp1, lbs1)

        histp_ref[pl.ds(t0, 1)] = ips0[None]
        hists_ref[pl.ds(t0, 1)] = iss0[None]
        histp_ref[pl.ds(t1, 1)] = ips1[None]
        hists_ref[pl.ds(t1, 1)] = iss1[None]
        return alls

    alls = alls0
    for pair_lo, pair_hi in ((0, 4), (4, 8), (8, 12), (12, 16), (16, 20),
                             (20, 25)):
        alls = lax.fori_loop(pair_lo, pair_hi,
                             functools.partial(body2, jmax=min(2 * pair_hi, S)),
                             alls)
    out_ref[0] = pacc_scr[...]


def _run_scan(pro_sm, skill_sm, change_sm, diffrow_sm, lo_r, na_r, np_r,
              ns_r, consts):
    row3 = pl.BlockSpec((S, _BB, D), lambda i: (0, i, 0))
    rowp = pl.BlockSpec((1, S, _BB), lambda i: (i, 0, 0))

    def full(a):
        return pl.BlockSpec(a.shape, lambda i: tuple(0 for _ in a.shape))

    return pl.pallas_call(
        _scan_kernel,
        grid=(_NB,),
        in_specs=[row3, row3, row3, row3, rowp, rowp, rowp, rowp]
                 + [full(c) for c in consts],
        out_specs=pl.BlockSpec((1, S, _BB), lambda i: (i, 0, 0)),
        out_shape=jax.ShapeDtypeStruct((_NB, S, _BB), jnp.float32),
        scratch_shapes=[pltpu.VMEM((S, D, _BB), jnp.float32),
                        pltpu.VMEM((S, D, _BB), jnp.float32),
                        pltpu.VMEM((S, _BB), jnp.int32),
                        pltpu.VMEM((S, _BB), jnp.int32),
                        pltpu.VMEM((S, _BB), jnp.float32)],
        compiler_params=pltpu.CompilerParams(
            dimension_semantics=("arbitrary",),
            vmem_limit_bytes=63 * 1024 * 1024),
    )(pro_sm, skill_sm, change_sm, diffrow_sm, lo_r, na_r, np_r, ns_r, *consts)


def _plane(arr_bs):
    return arr_bs.T.reshape(S, _NB, _BB).transpose(1, 0, 2)


def kernel(last_problem, last_skill, last_ans, next_problem, next_skill,
           next_ans, pro_embed, skill_embed, ans_embed, time_embed, ls_state,
           pro_state0, skill_state0, akt_pro_diff, akt_pro_change, W_out1,
           b_out1, W_out2, b_out2, W_pf, b_pf, W_ps, b_ps, W_af, b_af, W_sf,
           b_sf, W_ss, b_ss, W_as, b_as):
    npb = next_problem.reshape(last_problem.shape)
    nsb = next_skill.reshape(last_skill.shape)
    nab = next_ans.reshape(last_ans.shape)

    np_idx = npb.T.reshape(-1)
    ns_idx = nsb.T.reshape(-1)

    diff_mat = jnp.concatenate(
        [akt_pro_diff[:, 0], jnp.zeros((96,), jnp.float32)]).reshape(782, D)
    np_hi = lax.shift_right_logical(np_idx, 7)
    np_lo = lax.bitwise_and(np_idx, 127)

    pro_rows, skill_rows, change_rows, diff_rows = _sc_gather_all(
        pro_embed, skill_embed, akt_pro_change, diff_mat, np_idx, np_hi, ns_idx)

    pro_sm = pro_rows.reshape(S, B, D)
    skill_sm = skill_rows.reshape(S, B, D)
    change_sm = change_rows.reshape(S, B, D)
    diffrow_sm = diff_rows.reshape(S, B, D)
    lo_r = _plane(np_lo.reshape(S, B).T)
    na_r = _plane(nab).astype(jnp.float32)
    np_r = _plane(npb)
    ns_r = _plane(nsb)

    def tT(w):
        return jnp.transpose(w)

    consts = [
        tT(ans_embed), tT(time_embed), tT(ls_state),
        tT(pro_state0[0:1]), tT(skill_state0[0:1]),
        tT(W_pf[:D]), tT(W_pf[D:]), b_pf.reshape(D, 1),
        tT(W_sf[:D]), tT(W_sf[D:]), b_sf.reshape(D, 1),
        tT(W_af[:D]), tT(W_af[D:]), b_af.reshape(D, 1),
        tT(W_ps[:D]), tT(W_ps[D:]), b_ps.reshape(D, 1),
        tT(W_ss[:D]), tT(W_ss[D:]), b_ss.reshape(D, 1),
        tT(W_as[:D]), tT(W_as[D:]), b_as.reshape(D, 1),
        tT(W_out1[0:D]), tT(W_out1[D:2 * D]), tT(W_out1[2 * D:3 * D]),
        tT(W_out1[3 * D:]), b_out1.reshape(D, 1), W_out2, b_out2.reshape(1, 1),
    ]
    out = _run_scan(pro_sm, skill_sm, change_sm, diffrow_sm, lo_r, na_r,
                    np_r, ns_r, consts)
    return out.transpose(0, 2, 1).reshape(B, S)


# speedup vs baseline: 1.0167x; 1.0167x over previous
import functools

import jax
import jax.numpy as jnp
from jax import lax
from jax.experimental import pallas as pl
from jax.experimental.pallas import tpu as pltpu
from jax.experimental.pallas import tpu_sc as plsc

D = 128
S = 50
B = 1024
N = B * S

_NC = 2
_NS = 16
_NW = _NC * _NS
_BPW = N // _NW
_CH = 400
_NCHUNK = _BPW // _CH


def _sc_gather_all(pro_embed, skill_embed, change, diff_mat, np_idx, np_hi,
                   ns_idx):
    mesh = plsc.VectorSubcoreMesh(core_axis_name="c", subcore_axis_name="s")

    @functools.partial(
        pl.kernel,
        mesh=mesh,
        out_type=(
            jax.ShapeDtypeStruct((N, D), jnp.float32),
            jax.ShapeDtypeStruct((N, D), jnp.float32),
            jax.ShapeDtypeStruct((N, D), jnp.float32),
            jax.ShapeDtypeStruct((N, D), jnp.float32),
        ),
        scratch_types=[
            pltpu.VMEM((_BPW,), jnp.int32),
            pltpu.VMEM((_BPW,), jnp.int32),
            pltpu.VMEM((_CH, D), jnp.float32),
            pltpu.VMEM((_CH, D), jnp.float32),
            pltpu.SemaphoreType.DMA,
            pltpu.SemaphoreType.DMA,
        ],
    )
    def k(pro_hbm, skill_hbm, change_hbm, diff_hbm, npi_hbm, nphi_hbm, nsi_hbm,
          pro_out, skill_out, change_out, diff_out, idxp_v, idxs_v,
          rows0_v, rows1_v, sem0, sem1):
        wid = lax.axis_index("s") * _NC + lax.axis_index("c")
        base = wid * _BPW

        pltpu.sync_copy(npi_hbm.at[pl.ds(base, _BPW)], idxp_v)
        pltpu.sync_copy(nsi_hbm.at[pl.ds(base, _BPW)], idxs_v)

        work = []
        for ci in range(_NCHUNK):
            work.append((pro_hbm, idxp_v, pro_out, ci))
            work.append((skill_hbm, idxs_v, skill_out, ci))
            work.append((change_hbm, idxs_v, change_out, ci))
        bufs = (rows0_v, rows1_v)
        sems = (sem0, sem1)

        def start(i):
            tbl, idx, _, ci = work[i]
            pltpu.async_copy(tbl.at[idx.at[pl.ds(ci * _CH, _CH)]],
                             bufs[i % 2], sems[i % 2])

        start(0)
        for i in range(len(work)):
            if i + 1 < len(work):
                start(i + 1)
            tbl, idx, out, ci = work[i]
            pltpu.make_async_copy(tbl.at[idx.at[pl.ds(ci * _CH, _CH)]],
                                  bufs[i % 2], sems[i % 2]).wait()
            pltpu.sync_copy(bufs[i % 2], out.at[pl.ds(base + ci * _CH, _CH)])

        pltpu.sync_copy(nphi_hbm.at[pl.ds(base, _BPW)], idxp_v)
        for ci in range(_NCHUNK):
            off = ci * _CH
            idx_c = idxp_v.at[pl.ds(off, _CH)]
            pltpu.async_copy(diff_hbm.at[idx_c], bufs[ci % 2], sems[ci % 2])
            if ci > 0:
                poff = (ci - 1) * _CH
                pltpu.make_async_copy(
                    diff_hbm.at[idxp_v.at[pl.ds(poff, _CH)]],
                    bufs[(ci - 1) % 2], sems[(ci - 1) % 2]).wait()
                pltpu.sync_copy(bufs[(ci - 1) % 2],
                                diff_out.at[pl.ds(base + poff, _CH)])
        last = _NCHUNK - 1
        pltpu.make_async_copy(diff_hbm.at[idxp_v.at[pl.ds(last * _CH, _CH)]],
                              bufs[last % 2], sems[last % 2]).wait()
        pltpu.sync_copy(bufs[last % 2],
                        diff_out.at[pl.ds(base + last * _CH, _CH)])

    return k(pro_embed, skill_embed, change, diff_mat, np_idx, np_hi, ns_idx)


_BB = 128
_NB = B // _BB
_TPAD = 56


def _scan_kernel(pro_ref, skill_ref, change_ref, diffrow_ref, lo_ref, na_ref,
                 np_ref, ns_ref, aet_ref, tet_ref, lst_ref, ps0t_ref, ss0t_ref,
                 wpfa_ref, wpfb_ref, bpf_ref, wsfa_ref, wsfb_ref, bsf_ref,
                 wafa_ref, wafb_ref, baf_ref, wpsa_ref, wpsb_ref, bps_ref,
                 wssa_ref, wssb_ref, bss_ref, wasa_ref, wasb_ref, bas_ref,
                 w1a_ref, w1b_ref, w1c_ref, w1d_ref, b1_ref, w2_ref, b2_ref,
                 out_ref, histp_ref, hists_ref, lbp_scr, lbs_scr, pacc_scr):
    f32 = jnp.float32
    i32 = jnp.int32
    dot = functools.partial(jnp.dot, preferred_element_type=f32)

    np_all = np_ref[0]
    ns_all = ns_ref[0]

    jjj = lax.broadcasted_iota(i32, (S, S, _BB), 0)
    ttt = lax.broadcasted_iota(i32, (S, S, _BB), 1)
    eqp = (np_all[:, None, :] == np_all[None, :, :]) & (jjj < ttt)
    eqs = (ns_all[:, None, :] == ns_all[None, :, :]) & (jjj < ttt)
    lbp_scr[...] = jnp.max(jnp.where(eqp, jjj, 0), axis=0)
    lbs_scr[...] = jnp.max(jnp.where(eqs, jjj, 0), axis=0)

    tp_tab = dot(wpfb_ref[...], tet_ref[:, 0:_TPAD])
    ts_tab = dot(wsfb_ref[...], tet_ref[:, 0:_TPAD])
    caf = dot(wafb_ref[...], tet_ref[:, 1:2]) + baf_ref[...]

    a0 = aet_ref[:, 0:1]
    a1 = aet_ref[:, 1:2]

    jj_s1b = lax.broadcasted_iota(i32, (S, 1, _BB), 0)
    sub56 = lax.broadcasted_iota(i32, (_TPAD, _BB), 0)
    subd = lax.broadcasted_iota(i32, (D, _BB), 0)

    histp_ref[...] = jnp.zeros((S, D, _BB), f32)
    hists_ref[...] = jnp.zeros((S, D, _BB), f32)
    histp_ref[0] = jnp.broadcast_to(ps0t_ref[...], (D, _BB))
    hists_ref[0] = jnp.broadcast_to(ss0t_ref[...], (D, _BB))
    alls0 = jnp.broadcast_to(lst_ref[...], (D, _BB))

    def substep(t, alls, lbps, lbss, lbpt_row, lbst_row):
        ohp = (sub56 == (t - lbpt_row)).astype(f32)
        ohs = (sub56 == (t - lbst_row)).astype(f32)

        lbps = lbps * jax.nn.sigmoid(
            dot(wpfa_ref[...], lbps) + dot(tp_tab, ohp) + bpf_ref[...])
        lbss = lbss * jax.nn.sigmoid(
            dot(wsfa_ref[...], lbss) + dot(ts_tab, ohs) + bsf_ref[...])
        lbas = alls * jax.nn.sigmoid(dot(wafa_ref[...], alls) + caf)

        pro_t = jnp.transpose(pro_ref[pl.ds(t, 1)][0])
        skill_t = jnp.transpose(skill_ref[pl.ds(t, 1)][0])
        change_t = jnp.transpose(change_ref[pl.ds(t, 1)][0])
        drow_t = jnp.transpose(diffrow_ref[pl.ds(t, 1)][0])
        lo_row = lo_ref[pl.ds(0, 1), pl.ds(t, 1), :][0]
        diff_row = jnp.sum(jnp.where(subd == lo_row, drow_t, 0.0),
                           axis=0, keepdims=True)
        na_row = na_ref[pl.ds(0, 1), pl.ds(t, 1), :][0]
        npe = pro_t + skill_t + diff_row * change_t
        nx = npe + a0 + na_row * (a1 - a0)

        h = jax.nn.relu(dot(w1a_ref[...], lbas) + dot(w1b_ref[...], lbps)
                        + dot(w1c_ref[...], lbss) + dot(w1d_ref[...], npe)
                        + b1_ref[...])
        logit = jnp.sum(h * w2_ref[...], axis=0, keepdims=True) + b2_ref[...]
        pacc_scr[pl.ds(t, 1)] = jax.nn.sigmoid(logit)

        alls_new = lbas + jnp.tanh(
            dot(wasa_ref[...], lbas) + dot(wasb_ref[...], nx) + bas_ref[...])
        ips = lbps + jnp.tanh(
            dot(wpsa_ref[...], lbps) + dot(wpsb_ref[...], nx) + bps_ref[...])
        iss = lbss + jnp.tanh(
            dot(wssa_ref[...], lbss) + dot(wssb_ref[...], nx) + bss_ref[...])
        return alls_new, ips, iss

    def body(t, alls, jmax):
        lbp0 = lbp_scr[pl.ds(t, 1)]
        lbs0 = lbs_scr[pl.ds(t, 1)]
        m0p = (jj_s1b[0:jmax] == lbp0).astype(f32)
        m0s = (jj_s1b[0:jmax] == lbs0).astype(f32)
        lbps0 = jnp.sum(histp_ref[0:jmax] * m0p, axis=0)
        lbss0 = jnp.sum(hists_ref[0:jmax] * m0s, axis=0)

        alls, ips0, iss0 = substep(t, alls, lbps0, lbss0, lbp0, lbs0)

        histp_ref[pl.ds(t, 1)] = ips0[None]
        hists_ref[pl.ds(t, 1)] = iss0[None]
        return alls

    alls = alls0
    for seg_lo, seg_hi in ((0, 8), (8, 16), (16, 24), (24, 32), (32, 40),
                           (40, S)):
        alls = lax.fori_loop(seg_lo, seg_hi,
                             functools.partial(body, jmax=seg_hi), alls)
    out_ref[0] = pacc_scr[...]


def _run_scan(pro_sm, skill_sm, change_sm, diffrow_sm, lo_r, na_r, np_r,
              ns_r, consts):
    row3 = pl.BlockSpec((S, _BB, D), lambda i: (0, i, 0))
    rowp = pl.BlockSpec((1, S, _BB), lambda i: (i, 0, 0))

    def full(a):
        return pl.BlockSpec(a.shape, lambda i: tuple(0 for _ in a.shape))

    return pl.pallas_call(
        _scan_kernel,
        grid=(_NB,),
        in_specs=[row3, row3, row3, row3, rowp, rowp, rowp, rowp]
                 + [full(c) for c in consts],
        out_specs=pl.BlockSpec((1, S, _BB), lambda i: (i, 0, 0)),
        out_shape=jax.ShapeDtypeStruct((_NB, S, _BB), jnp.float32),
        scratch_shapes=[pltpu.VMEM((S, D, _BB), jnp.float32),
                        pltpu.VMEM((S, D, _BB), jnp.float32),
                        pltpu.VMEM((S, _BB), jnp.int32),
                        pltpu.VMEM((S, _BB), jnp.int32),
                        pltpu.VMEM((S, _BB), jnp.float32)],
        compiler_params=pltpu.CompilerParams(
            dimension_semantics=("arbitrary",),
            vmem_limit_bytes=63 * 1024 * 1024),
    )(pro_sm, skill_sm, change_sm, diffrow_sm, lo_r, na_r, np_r, ns_r, *consts)


def _plane(arr_bs):
    return arr_bs.T.reshape(S, _NB, _BB).transpose(1, 0, 2)


def kernel(last_problem, last_skill, last_ans, next_problem, next_skill,
           next_ans, pro_embed, skill_embed, ans_embed, time_embed, ls_state,
           pro_state0, skill_state0, akt_pro_diff, akt_pro_change, W_out1,
           b_out1, W_out2, b_out2, W_pf, b_pf, W_ps, b_ps, W_af, b_af, W_sf,
           b_sf, W_ss, b_ss, W_as, b_as):
    npb = next_problem.reshape(last_problem.shape)
    nsb = next_skill.reshape(last_skill.shape)
    nab = next_ans.reshape(last_ans.shape)

    np_idx = npb.T.reshape(-1)
    ns_idx = nsb.T.reshape(-1)

    diff_mat = jnp.concatenate(
        [akt_pro_diff[:, 0], jnp.zeros((96,), jnp.float32)]).reshape(782, D)
    np_hi = lax.shift_right_logical(np_idx, 7)
    np_lo = lax.bitwise_and(np_idx, 127)

    pro_rows, skill_rows, change_rows, diff_rows = _sc_gather_all(
        pro_embed, skill_embed, akt_pro_change, diff_mat, np_idx, np_hi, ns_idx)

    pro_sm = pro_rows.reshape(S, B, D)
    skill_sm = skill_rows.reshape(S, B, D)
    change_sm = change_rows.reshape(S, B, D)
    diffrow_sm = diff_rows.reshape(S, B, D)
    lo_r = _plane(np_lo.reshape(S, B).T)
    na_r = _plane(nab).astype(jnp.float32)
    np_r = _plane(npb)
    ns_r = _plane(nsb)

    def tT(w):
        return jnp.transpose(w)

    consts = [
        tT(ans_embed), tT(time_embed), tT(ls_state),
        tT(pro_state0[0:1]), tT(skill_state0[0:1]),
        tT(W_pf[:D]), tT(W_pf[D:]), b_pf.reshape(D, 1),
        tT(W_sf[:D]), tT(W_sf[D:]), b_sf.reshape(D, 1),
        tT(W_af[:D]), tT(W_af[D:]), b_af.reshape(D, 1),
        tT(W_ps[:D]), tT(W_ps[D:]), b_ps.reshape(D, 1),
        tT(W_ss[:D]), tT(W_ss[D:]), b_ss.reshape(D, 1),
        tT(W_as[:D]), tT(W_as[D:]), b_as.reshape(D, 1),
        tT(W_out1[0:D]), tT(W_out1[D:2 * D]), tT(W_out1[2 * D:3 * D]),
        tT(W_out1[3 * D:]), b_out1.reshape(D, 1), W_out2, b_out2.reshape(1, 1),
    ]
    out = _run_scan(pro_sm, skill_sm, change_sm, diffrow_sm, lo_r, na_r,
                    np_r, ns_r, consts)
    return out.transpose(0, 2, 1).reshape(B, S)


# speedup vs baseline: 1.1136x; 1.0952x over previous
import functools

import jax
import jax.numpy as jnp
from jax import lax
from jax.experimental import pallas as pl
from jax.experimental.pallas import tpu as pltpu
from jax.experimental.pallas import tpu_sc as plsc

D = 128
S = 50
B = 1024
N = B * S

_NC = 2
_NS = 16
_NW = _NC * _NS
_BPW = N // _NW
_CH = 400
_NCHUNK = _BPW // _CH


def _sc_gather_all(pro_embed, skill_embed, change, diff_mat, np_idx, np_hi,
                   ns_idx):
    n = np_idx.shape[0]
    bpw = n // _NW
    nchunk = bpw // _CH
    mesh = plsc.VectorSubcoreMesh(core_axis_name="c", subcore_axis_name="s")

    @functools.partial(
        pl.kernel,
        mesh=mesh,
        out_type=(
            jax.ShapeDtypeStruct((n, D), jnp.float32),
            jax.ShapeDtypeStruct((n, D), jnp.float32),
            jax.ShapeDtypeStruct((n, D), jnp.float32),
            jax.ShapeDtypeStruct((n, D), jnp.float32),
        ),
        scratch_types=[
            pltpu.VMEM((bpw,), jnp.int32),
            pltpu.VMEM((bpw,), jnp.int32),
            pltpu.VMEM((_CH, D), jnp.float32),
            pltpu.VMEM((_CH, D), jnp.float32),
            pltpu.SemaphoreType.DMA,
            pltpu.SemaphoreType.DMA,
        ],
    )
    def k(pro_hbm, skill_hbm, change_hbm, diff_hbm, npi_hbm, nphi_hbm, nsi_hbm,
          pro_out, skill_out, change_out, diff_out, idxp_v, idxs_v,
          rows0_v, rows1_v, sem0, sem1):
        wid = lax.axis_index("s") * _NC + lax.axis_index("c")
        base = wid * bpw

        pltpu.sync_copy(npi_hbm.at[pl.ds(base, bpw)], idxp_v)
        pltpu.sync_copy(nsi_hbm.at[pl.ds(base, bpw)], idxs_v)

        work = []
        for ci in range(nchunk):
            work.append((pro_hbm, idxp_v, pro_out, ci))
            work.append((skill_hbm, idxs_v, skill_out, ci))
            work.append((change_hbm, idxs_v, change_out, ci))
        bufs = (rows0_v, rows1_v)
        sems = (sem0, sem1)

        def start(i):
            tbl, idx, _, ci = work[i]
            pltpu.async_copy(tbl.at[idx.at[pl.ds(ci * _CH, _CH)]],
                             bufs[i % 2], sems[i % 2])

        start(0)
        for i in range(len(work)):
            if i + 1 < len(work):
                start(i + 1)
            tbl, idx, out, ci = work[i]
            pltpu.make_async_copy(tbl.at[idx.at[pl.ds(ci * _CH, _CH)]],
                                  bufs[i % 2], sems[i % 2]).wait()
            pltpu.sync_copy(bufs[i % 2], out.at[pl.ds(base + ci * _CH, _CH)])

        pltpu.sync_copy(nphi_hbm.at[pl.ds(base, bpw)], idxp_v)
        for ci in range(nchunk):
            off = ci * _CH
            idx_c = idxp_v.at[pl.ds(off, _CH)]
            pltpu.async_copy(diff_hbm.at[idx_c], bufs[ci % 2], sems[ci % 2])
            if ci > 0:
                poff = (ci - 1) * _CH
                pltpu.make_async_copy(
                    diff_hbm.at[idxp_v.at[pl.ds(poff, _CH)]],
                    bufs[(ci - 1) % 2], sems[(ci - 1) % 2]).wait()
                pltpu.sync_copy(bufs[(ci - 1) % 2],
                                diff_out.at[pl.ds(base + poff, _CH)])
        last = nchunk - 1
        pltpu.make_async_copy(diff_hbm.at[idxp_v.at[pl.ds(last * _CH, _CH)]],
                              bufs[last % 2], sems[last % 2]).wait()
        pltpu.sync_copy(bufs[last % 2],
                        diff_out.at[pl.ds(base + last * _CH, _CH)])

    return k(pro_embed, skill_embed, change, diff_mat, np_idx, np_hi, ns_idx)


_BB = 128
_NB = B // _BB
_TPAD = 56


def _scan_kernel(pro_ref, skill_ref, change_ref, diffrow_ref, lo_ref, na_ref,
                 np_ref, ns_ref, aet_ref, tet_ref, lst_ref, ps0t_ref, ss0t_ref,
                 wpfa_ref, wpfb_ref, bpf_ref, wsfa_ref, wsfb_ref, bsf_ref,
                 wafa_ref, wafb_ref, baf_ref, wpsa_ref, wpsb_ref, bps_ref,
                 wssa_ref, wssb_ref, bss_ref, wasa_ref, wasb_ref, bas_ref,
                 w1a_ref, w1b_ref, w1c_ref, w1d_ref, b1_ref, w2_ref, b2_ref,
                 out_ref, histp_ref, hists_ref, lbp_scr, lbs_scr, pacc_scr):
    f32 = jnp.float32
    i32 = jnp.int32
    dot = functools.partial(jnp.dot, preferred_element_type=f32)

    np_all = np_ref[0]
    ns_all = ns_ref[0]

    jjj = lax.broadcasted_iota(i32, (S, S, _BB), 0)
    ttt = lax.broadcasted_iota(i32, (S, S, _BB), 1)
    eqp = (np_all[:, None, :] == np_all[None, :, :]) & (jjj < ttt)
    eqs = (ns_all[:, None, :] == ns_all[None, :, :]) & (jjj < ttt)
    lbp_scr[...] = jnp.max(jnp.where(eqp, jjj, 0), axis=0)
    lbs_scr[...] = jnp.max(jnp.where(eqs, jjj, 0), axis=0)

    tp_tab = dot(wpfb_ref[...], tet_ref[:, 0:_TPAD])
    ts_tab = dot(wsfb_ref[...], tet_ref[:, 0:_TPAD])
    caf = dot(wafb_ref[...], tet_ref[:, 1:2]) + baf_ref[...]

    a0 = aet_ref[:, 0:1]
    a1 = aet_ref[:, 1:2]

    jj_s1b = lax.broadcasted_iota(i32, (S, 1, _BB), 0)
    sub56 = lax.broadcasted_iota(i32, (_TPAD, _BB), 0)
    subd = lax.broadcasted_iota(i32, (D, _BB), 0)

    histp_ref[...] = jnp.zeros((S, D, _BB), f32)
    hists_ref[...] = jnp.zeros((S, D, _BB), f32)
    histp_ref[0] = jnp.broadcast_to(ps0t_ref[...], (D, _BB))
    hists_ref[0] = jnp.broadcast_to(ss0t_ref[...], (D, _BB))
    alls0 = jnp.broadcast_to(lst_ref[...], (D, _BB))

    def substep(t, alls, lbps, lbss, lbpt_row, lbst_row):
        ohp = (sub56 == (t - lbpt_row)).astype(f32)
        ohs = (sub56 == (t - lbst_row)).astype(f32)

        lbps = lbps * jax.nn.sigmoid(
            dot(wpfa_ref[...], lbps) + dot(tp_tab, ohp) + bpf_ref[...])
        lbss = lbss * jax.nn.sigmoid(
            dot(wsfa_ref[...], lbss) + dot(ts_tab, ohs) + bsf_ref[...])
        lbas = alls * jax.nn.sigmoid(dot(wafa_ref[...], alls) + caf)

        pro_t = jnp.transpose(pro_ref[pl.ds(t, 1)][0])
        skill_t = jnp.transpose(skill_ref[pl.ds(t, 1)][0])
        change_t = jnp.transpose(change_ref[pl.ds(t, 1)][0])
        drow_t = jnp.transpose(diffrow_ref[pl.ds(t, 1)][0])
        lo_row = lo_ref[pl.ds(0, 1), pl.ds(t, 1), :][0]
        diff_row = jnp.sum(jnp.where(subd == lo_row, drow_t, 0.0),
                           axis=0, keepdims=True)
        na_row = na_ref[pl.ds(0, 1), pl.ds(t, 1), :][0]
        npe = pro_t + skill_t + diff_row * change_t
        nx = npe + a0 + na_row * (a1 - a0)

        h = jax.nn.relu(dot(w1a_ref[...], lbas) + dot(w1b_ref[...], lbps)
                        + dot(w1c_ref[...], lbss) + dot(w1d_ref[...], npe)
                        + b1_ref[...])
        logit = jnp.sum(h * w2_ref[...], axis=0, keepdims=True) + b2_ref[...]
        pacc_scr[pl.ds(t, 1)] = jax.nn.sigmoid(logit)

        alls_new = lbas + jnp.tanh(
            dot(wasa_ref[...], lbas) + dot(wasb_ref[...], nx) + bas_ref[...])
        ips = lbps + jnp.tanh(
            dot(wpsa_ref[...], lbps) + dot(wpsb_ref[...], nx) + bps_ref[...])
        iss = lbss + jnp.tanh(
            dot(wssa_ref[...], lbss) + dot(wssb_ref[...], nx) + bss_ref[...])
        return alls_new, ips, iss

    def body(t, alls, jmax):
        lbp0 = lbp_scr[pl.ds(t, 1)]
        lbs0 = lbs_scr[pl.ds(t, 1)]
        m0p = (jj_s1b[0:jmax] == lbp0).astype(f32)
        m0s = (jj_s1b[0:jmax] == lbs0).astype(f32)
        lbps0 = jnp.sum(histp_ref[0:jmax] * m0p, axis=0)
        lbss0 = jnp.sum(hists_ref[0:jmax] * m0s, axis=0)

        alls, ips0, iss0 = substep(t, alls, lbps0, lbss0, lbp0, lbs0)

        histp_ref[pl.ds(t, 1)] = ips0[None]
        hists_ref[pl.ds(t, 1)] = iss0[None]
        return alls

    alls = alls0
    for seg_lo, seg_hi in ((0, 8), (8, 16), (16, 24), (24, 32), (32, 40),
                           (40, S)):
        alls = lax.fori_loop(seg_lo, seg_hi,
                             functools.partial(body, jmax=seg_hi), alls)
    out_ref[0] = pacc_scr[...]


def _run_scan(pro_sm, skill_sm, change_sm, diffrow_sm, lo_r, na_r, np_r,
              ns_r, consts):
    row3 = pl.BlockSpec((S, _BB, D), lambda i: (0, i, 0))
    rowp = pl.BlockSpec((1, S, _BB), lambda i: (i, 0, 0))

    def full(a):
        return pl.BlockSpec(a.shape, lambda i: tuple(0 for _ in a.shape))

    nb = pro_sm.shape[1] // _BB
    return pl.pallas_call(
        _scan_kernel,
        grid=(nb,),
        in_specs=[row3, row3, row3, row3, rowp, rowp, rowp, rowp]
                 + [full(c) for c in consts],
        out_specs=pl.BlockSpec((1, S, _BB), lambda i: (i, 0, 0)),
        out_shape=jax.ShapeDtypeStruct((nb, S, _BB), jnp.float32),
        scratch_shapes=[pltpu.VMEM((S, D, _BB), jnp.float32),
                        pltpu.VMEM((S, D, _BB), jnp.float32),
                        pltpu.VMEM((S, _BB), jnp.int32),
                        pltpu.VMEM((S, _BB), jnp.int32),
                        pltpu.VMEM((S, _BB), jnp.float32)],
        compiler_params=pltpu.CompilerParams(
            dimension_semantics=("arbitrary",),
            vmem_limit_bytes=63 * 1024 * 1024),
    )(pro_sm, skill_sm, change_sm, diffrow_sm, lo_r, na_r, np_r, ns_r, *consts)


def _plane(arr_bs):
    return arr_bs.T.reshape(S, _NB, _BB).transpose(1, 0, 2)


def kernel(last_problem, last_skill, last_ans, next_problem, next_skill,
           next_ans, pro_embed, skill_embed, ans_embed, time_embed, ls_state,
           pro_state0, skill_state0, akt_pro_diff, akt_pro_change, W_out1,
           b_out1, W_out2, b_out2, W_pf, b_pf, W_ps, b_ps, W_af, b_af, W_sf,
           b_sf, W_ss, b_ss, W_as, b_as):
    npb = next_problem.reshape(last_problem.shape)
    nsb = next_skill.reshape(last_skill.shape)
    nab = next_ans.reshape(last_ans.shape)

    diff_mat = jnp.concatenate(
        [akt_pro_diff[:, 0], jnp.zeros((96,), jnp.float32)]).reshape(782, D)

    def tT(w):
        return jnp.transpose(w)

    consts = [
        tT(ans_embed), tT(time_embed), tT(ls_state),
        tT(pro_state0[0:1]), tT(skill_state0[0:1]),
        tT(W_pf[:D]), tT(W_pf[D:]), b_pf.reshape(D, 1),
        tT(W_sf[:D]), tT(W_sf[D:]), b_sf.reshape(D, 1),
        tT(W_af[:D]), tT(W_af[D:]), b_af.reshape(D, 1),
        tT(W_ps[:D]), tT(W_ps[D:]), b_ps.reshape(D, 1),
        tT(W_ss[:D]), tT(W_ss[D:]), b_ss.reshape(D, 1),
        tT(W_as[:D]), tT(W_as[D:]), b_as.reshape(D, 1),
        tT(W_out1[0:D]), tT(W_out1[D:2 * D]), tT(W_out1[2 * D:3 * D]),
        tT(W_out1[3 * D:]), b_out1.reshape(D, 1), W_out2, b_out2.reshape(1, 1),
    ]

    outs = []
    half = B // 2
    for h in range(2):
        sl = slice(h * half, (h + 1) * half)
        npb_h, nsb_h, nab_h = npb[sl], nsb[sl], nab[sl]
        np_idx = npb_h.T.reshape(-1)
        ns_idx = nsb_h.T.reshape(-1)
        np_hi = lax.shift_right_logical(np_idx, 7)
        np_lo = lax.bitwise_and(np_idx, 127)

        pro_rows, skill_rows, change_rows, diff_rows = _sc_gather_all(
            pro_embed, skill_embed, akt_pro_change, diff_mat,
            np_idx, np_hi, ns_idx)

        nb = half // _BB

        def plane(arr_bs):
            return arr_bs.T.reshape(S, nb, _BB).transpose(1, 0, 2)

        out = _run_scan(
            pro_rows.reshape(S, half, D), skill_rows.reshape(S, half, D),
            change_rows.reshape(S, half, D), diff_rows.reshape(S, half, D),
            plane(np_lo.reshape(S, half).T), plane(nab_h).astype(jnp.float32),
            plane(npb_h), plane(nsb_h), consts)
        outs.append(out.transpose(0, 2, 1).reshape(half, S))
    return jnp.concatenate(outs, axis=0)
